# Initial kernel scaffold; baseline (speedup 1.0000x reference)
import jax
import jax.numpy as jnp
from jax.experimental import pallas as pl


def kernel(hidden, W1, b1, ln_g, ln_b, W2, b2, Wp, bp, working_embeddings, working_rewards, longterm_embeddings, longterm_rewards):
    raise NotImplementedError("write your pallas kernel here")


# speedup vs baseline: 4.0781x; 4.0781x over previous
import functools

import jax
import jax.numpy as jnp
from jax import lax
from jax.experimental import pallas as pl
from jax.experimental.pallas import tpu as pltpu
from jax.experimental.pallas import tpu_sc as plsc

B = 256
H = 1024
D = 64
K = 16
SUB = 128
MW = 4096
ML = 131072
TILE = 4096
NSUB_W = MW // SUB
NSUB_L = ML // SUB
ALPHA = 0.1
DOPA = 0.5

NC, NS = 2, 16
NWORK = NC * NS
CHUNK = (B * K) // NWORK


def _qnet_body(hid_ref, w1_ref, b1_ref, g_ref, bb_ref, w2_ref, b2_ref,
               qb_ref, q2_ref):
    h = jnp.dot(hid_ref[...], w1_ref[...],
                preferred_element_type=jnp.float32) + b1_ref[...]
    mu = jnp.mean(h, axis=-1, keepdims=True)
    var = jnp.mean((h - mu) ** 2, axis=-1, keepdims=True)
    h = g_ref[...] * (h - mu) / jnp.sqrt(var + 1e-5) + bb_ref[...]
    h = 0.5 * h * (1.0 + lax.erf(h / jnp.sqrt(2.0).astype(h.dtype)))
    q = jnp.dot(h, w2_ref[...], preferred_element_type=jnp.float32) + b2_ref[...]
    n = jnp.sqrt(jnp.sum(q * q, axis=-1, keepdims=True))
    n = jnp.maximum(n, 1e-9)
    qb = jnp.tanh(n) * q / n
    qb_ref[...] = qb
    q2_ref[...] = jnp.sum(qb * qb, axis=-1, keepdims=True)


def _qnet(hidden, W1, b1, ln_g, ln_b, W2, b2):
    return pl.pallas_call(
        _qnet_body,
        out_shape=[jax.ShapeDtypeStruct((B, D), jnp.float32),
                   jax.ShapeDtypeStruct((B, 1), jnp.float32)],
    )(hidden, W1, b1, ln_g, ln_b, W2, b2)


def _key_body(qb_ref, q2_ref, m_ref, keys_ref, mins_ref):
    m = m_ref[...]
    n = jnp.sqrt(jnp.sum(m * m, axis=-1, keepdims=True))
    n = jnp.maximum(n, 1e-9)
    mb = jnp.tanh(n) * m / n
    m2b = jnp.sum(mb * mb, axis=-1)[None, :]
    qmb = lax.dot_general(qb_ref[...], mb, (((1,), (1,)), ((), ())),
                          preferred_element_type=jnp.float32)
    q2 = q2_ref[...]
    sqd = jnp.maximum(q2 + m2b - 2.0 * qmb, 0.0)
    denom = jnp.maximum((1.0 - q2) * (1.0 - m2b), 1e-9)
    keys = 1.0 + 2.0 * sqd / denom
    keys_ref[...] = keys
    mins_ref[...] = jnp.min(keys.reshape(B, TILE // SUB, SUB), axis=2)[None]


def _keys(qb, q2, emb):
    m = emb.shape[0]
    nt = m // TILE
    return pl.pallas_call(
        _key_body,
        grid=(nt,),
        in_specs=[pl.BlockSpec((B, D), lambda i: (0, 0)),
                  pl.BlockSpec((B, 1), lambda i: (0, 0)),
                  pl.BlockSpec((TILE, D), lambda i: (i, 0))],
        out_specs=[pl.BlockSpec((B, TILE), lambda i: (0, i)),
                   pl.BlockSpec((1, B, TILE // SUB), lambda i: (i, 0, 0))],
        out_shape=[jax.ShapeDtypeStruct((B, m), jnp.float32),
                   jax.ShapeDtypeStruct((nt, B, TILE // SUB), jnp.float32)],
    )(qb, q2, emb)


def _extract_topk(x, payloads):
    n = x.shape[1]
    idx2 = lax.broadcasted_iota(jnp.int32, (B, n), 1)
    outs = [[] for _ in payloads]
    for _ in range(K):
        mv = jnp.min(x, axis=1, keepdims=True)
        am = jnp.min(jnp.where(x == mv, idx2, n), axis=1, keepdims=True)
        hit = idx2 == am
        for j, p in enumerate(payloads):
            outs[j].append(jnp.sum(jnp.where(hit, p, 0), axis=1, keepdims=True))
        x = jnp.where(hit, jnp.inf, x)
    return [jnp.concatenate(o, axis=1) for o in outs]


def _phase1_body(minsl_ref, minsw_ref, flatl_ref, flatw_ref):
    rowbase = lax.broadcasted_iota(jnp.int32, (B, K), 0)
    for mins_ref, nsub, out_ref in ((minsl_ref, NSUB_L, flatl_ref),
                                    (minsw_ref, NSUB_W, flatw_ref)):
        x = mins_ref[...]
        subidx = lax.broadcasted_iota(jnp.int32, (B, nsub), 1)
        (subs,) = _extract_topk(x, [subidx])
        out_ref[...] = subs + rowbase * nsub


def _phase1(mins_l, mins_w):
    return pl.pallas_call(
        _phase1_body,
        out_shape=[jax.ShapeDtypeStruct((B, K), jnp.int32),
                   jax.ShapeDtypeStruct((B, K), jnp.int32)],
    )(mins_l, mins_w)


def _sc_gather_keys(keys_l_rows, keys_w_rows, flat_l, flat_w):
    mesh = plsc.VectorSubcoreMesh(core_axis_name="c", subcore_axis_name="s")

    @functools.partial(
        pl.kernel, mesh=mesh,
        out_type=[jax.ShapeDtypeStruct((B * K, SUB), jnp.float32),
                  jax.ShapeDtypeStruct((B * K, SUB), jnp.float32)],
        scratch_types=[pltpu.VMEM((CHUNK,), jnp.int32),
                       pltpu.VMEM((CHUNK, SUB), jnp.float32),
                       pltpu.SemaphoreType.DMA],
    )
    def k(kl_hbm, kw_hbm, il_hbm, iw_hbm, ol_hbm, ow_hbm, idx_v, rows_v, sem):
        wid = lax.axis_index("s") * NC + lax.axis_index("c")
        base = wid * CHUNK
        pltpu.sync_copy(il_hbm.at[pl.ds(base, CHUNK)], idx_v)
        pltpu.async_copy(kl_hbm.at[idx_v], rows_v, sem).wait()
        pltpu.sync_copy(rows_v, ol_hbm.at[pl.ds(base, CHUNK)])
        pltpu.sync_copy(iw_hbm.at[pl.ds(base, CHUNK)], idx_v)
        pltpu.async_copy(kw_hbm.at[idx_v], rows_v, sem).wait()
        pltpu.sync_copy(rows_v, ow_hbm.at[pl.ds(base, CHUNK)])

    return k(keys_l_rows, keys_w_rows, flat_l, flat_w)


def _phase2_body(candl_ref, flatl_ref, candw_ref, flatw_ref,
                 gl_ref, rl_ref, gw_ref, rw_ref):
    rowbase = lax.broadcasted_iota(jnp.int32, (B, K), 0)
    colmod = lax.broadcasted_iota(jnp.int32, (B, K * SUB), 1) & (SUB - 1)
    for cand_ref, flat_ref, nsub, g_ref, r_ref in (
            (candl_ref, flatl_ref, NSUB_L, gl_ref, rl_ref),
            (candw_ref, flatw_ref, NSUB_W, gw_ref, rw_ref)):
        sub = flat_ref[...] - rowbase * nsub
        base = jnp.broadcast_to((sub * SUB)[:, :, None], (B, K, SUB))
        gmat = base.reshape(B, K * SUB) + colmod
        (gidx,) = _extract_topk(cand_ref[...], [gmat])
        g_ref[...] = gidx
        r_ref[...] = lax.shift_right_logical(gidx, 4)


def _phase2(cand_l, flat_l, cand_w, flat_w):
    return pl.pallas_call(
        _phase2_body,
        out_shape=[jax.ShapeDtypeStruct((B, K), jnp.int32)] * 4,
    )(cand_l, flat_l, cand_w, flat_w)


def _sc_gather_rows(emb_l, emb_w, rew_l_rows, rew_w_rows,
                    gidx_l, ridx_l, gidx_w, ridx_w):
    mesh = plsc.VectorSubcoreMesh(core_axis_name="c", subcore_axis_name="s")

    @functools.partial(
        pl.kernel, mesh=mesh,
        compiler_params=pltpu.CompilerParams(use_tc_tiling_on_sc=False),
        out_type=[jax.ShapeDtypeStruct((B * K, D), jnp.float32),
                  jax.ShapeDtypeStruct((B * K, K), jnp.float32),
                  jax.ShapeDtypeStruct((B * K, D), jnp.float32),
                  jax.ShapeDtypeStruct((B * K, K), jnp.float32)],
        scratch_types=[pltpu.VMEM((CHUNK,), jnp.int32),
                       pltpu.VMEM((CHUNK, D), jnp.float32),
                       pltpu.VMEM((CHUNK, K), jnp.float32),
                       pltpu.SemaphoreType.DMA],
    )
    def k(el_hbm, ew_hbm, rl_hbm, rw_hbm, gl_hbm, cl_hbm, gw_hbm, cw_hbm,
          oel_hbm, orl_hbm, oew_hbm, orw_hbm, idx_v, erow_v, rrow_v, sem):
        wid = lax.axis_index("s") * NC + lax.axis_index("c")
        base = wid * CHUNK
        for tbl, idx_hbm, buf, out_hbm in ((el_hbm, gl_hbm, erow_v, oel_hbm),
                                           (rl_hbm, cl_hbm, rrow_v, orl_hbm),
                                           (ew_hbm, gw_hbm, erow_v, oew_hbm),
                                           (rw_hbm, cw_hbm, rrow_v, orw_hbm)):
            pltpu.sync_copy(idx_hbm.at[pl.ds(base, CHUNK)], idx_v)
            pltpu.async_copy(tbl.at[idx_v], buf, sem).wait()
            pltpu.sync_copy(buf, out_hbm.at[pl.ds(base, CHUNK)])

    return k(emb_l, emb_w, rew_l_rows, rew_w_rows, gidx_l, ridx_l,
             gidx_w, ridx_w)


def _tier_retrieve(qb, q2, rows, rrows, gidx):
    m = rows.reshape(B, K, D)
    n = jnp.sqrt(jnp.sum(m * m, axis=-1, keepdims=True))
    n = jnp.maximum(n, 1e-9)
    mb = jnp.tanh(n) * m / n
    m2b = jnp.sum(mb * mb, axis=-1)
    qm = jnp.sum(qb[:, None, :] * mb, axis=-1)
    sqd = jnp.maximum(q2 + m2b - 2.0 * qm, 0.0)
    denom = jnp.maximum((1.0 - q2) * (1.0 - m2b), 1e-9)
    arg = jnp.maximum(1.0 + 2.0 * sqd / denom, 1.0 + 1e-7)
    vals = -jnp.log(arg + jnp.sqrt((arg + 1.0) * (arg - 1.0)))
    rr = rrows.reshape(B, K, K)
    off = gidx & (K - 1)
    hit = lax.broadcasted_iota(jnp.int32, (B, K, K), 2) == off[:, :, None]
    rk = jnp.sum(jnp.where(hit, rr, 0.0), axis=-1)
    logits = vals * (1.0 + DOPA * jnp.tanh(rk))
    attn = jax.nn.softmax(logits, axis=-1)
    out = jnp.sum(attn[:, :, None] * m, axis=1)
    score = jnp.mean(vals, axis=-1, keepdims=True)
    return out, score


def _final_body(hid_ref, qb_ref, q2_ref, rw_rows_ref, rw_rr_ref, gw_ref,
                rl_rows_ref, rl_rr_ref, gl_ref, wp_ref, bp_ref, out_ref):
    qb = qb_ref[...]
    q2 = q2_ref[...]
    rw, sw = _tier_retrieve(qb, q2, rw_rows_ref[...], rw_rr_ref[...], gw_ref[...])
    rl, sl = _tier_retrieve(qb, q2, rl_rows_ref[...], rl_rr_ref[...], gl_ref[...])
    gate = jax.nn.sigmoid(sw - sl)
    blended = gate * rw + (1.0 - gate) * rl
    proj = jnp.dot(blended, wp_ref[...],
                   preferred_element_type=jnp.float32) + bp_ref[...]
    out_ref[...] = hid_ref[...] + ALPHA * proj


def _final(hidden, qb, q2, rows_w, rr_w, gidx_w, rows_l, rr_l, gidx_l, Wp, bp):
    return pl.pallas_call(
        _final_body,
        out_shape=jax.ShapeDtypeStruct((B, H), jnp.float32),
    )(hidden, qb, q2, rows_w, rr_w, gidx_w, rows_l, rr_l, gidx_l, Wp, bp)


def kernel(hidden, W1, b1, ln_g, ln_b, W2, b2, Wp, bp,
           working_embeddings, working_rewards,
           longterm_embeddings, longterm_rewards):
    qb, q2 = _qnet(hidden, W1, b1.reshape(1, D), ln_g.reshape(1, D),
                   ln_b.reshape(1, D), W2, b2.reshape(1, D))
    keys_l, mins_l = _keys(qb, q2, longterm_embeddings)
    keys_w, mins_w = _keys(qb, q2, working_embeddings)
    mins_l = mins_l.transpose(1, 0, 2).reshape(B, NSUB_L)
    mins_w = mins_w.transpose(1, 0, 2).reshape(B, NSUB_W)
    flat_l, flat_w = _phase1(mins_l, mins_w)
    cand_l, cand_w = _sc_gather_keys(
        keys_l.reshape(B * NSUB_L, SUB), keys_w.reshape(B * NSUB_W, SUB),
        flat_l.reshape(B * K), flat_w.reshape(B * K))
    gidx_l, ridx_l, gidx_w, ridx_w = _phase2(
        cand_l.reshape(B, K * SUB), flat_l, cand_w.reshape(B, K * SUB), flat_w)
    rows_l, rr_l, rows_w, rr_w = _sc_gather_rows(
        longterm_embeddings, working_embeddings,
        longterm_rewards.reshape(ML // K, K), working_rewards.reshape(MW // K, K),
        gidx_l.reshape(B * K), ridx_l.reshape(B * K),
        gidx_w.reshape(B * K), ridx_w.reshape(B * K))
    return _final(hidden, qb, q2, rows_w, rr_w, gidx_w,
                  rows_l, rr_l, gidx_l, Wp, bp.reshape(1, H))
